# baseline (speedup 1.0000x reference)
import functools

import jax
import jax.numpy as jnp
from jax import lax
from jax.experimental import pallas as pl
from jax.experimental.pallas import tpu as pltpu
from jax.experimental.pallas import tpu_sc as plsc

N = 10000
E = 320000
D = 128
NP = 10240
NC = 2
NS = 16
NW = NC * NS
EPW = E // NW
EPWP = 10240
K = 128
G = EPWP // K
PH = 2
GP = G // PH
RPW = NP // NS
DW = 16
DC = 64
MB = 2000


def _segsum_body(h_hbm, src_hbm, dst_hbm, zrow_hbm, out_hbm,
                 idx_s, idx_d, idx1s, idx1d, rows_a, acc):
    cid = lax.axis_index("c")
    sid = lax.axis_index("s")
    wid = cid * NS + sid
    r0 = sid * RPW

    pltpu.sync_copy(zrow_hbm, acc.at[pl.ds(r0, RPW)])
    plsc.subcore_barrier()

    for ph in range(PH):
        pltpu.sync_copy(src_hbm.at[wid, pl.ds(ph * GP, GP)], idx_s)
        pltpu.sync_copy(dst_hbm.at[wid, pl.ds(ph * GP, GP)], idx_d)

        @pl.loop(0, GP)
        def _(g):
            for c in range(K // 16):
                idx1s[pl.ds(16 * c, 16)] = idx_s[g, pl.ds(16 * c, 16)]
                idx1d[pl.ds(16 * c, 16)] = idx_d[g, pl.ds(16 * c, 16)]
            pltpu.sync_copy(h_hbm.at[idx1s], rows_a)
            pltpu.sync_copy(rows_a, acc.at[idx1d], add=True)

    plsc.subcore_barrier()
    pltpu.sync_copy(acc.at[pl.ds(r0, RPW)], out_hbm.at[cid, pl.ds(r0, RPW)])


def _deg_body(dst_hbm, ones_hbm, zrow_hbm, out_hbm, idx_d, ones_v, acc):
    cid = lax.axis_index("c")
    sid = lax.axis_index("s")
    wid = cid * NS + sid
    r0 = sid * RPW

    pltpu.sync_copy(zrow_hbm, acc.at[pl.ds(r0, RPW)])
    pltpu.sync_copy(ones_hbm, ones_v)
    plsc.subcore_barrier()

    for ph in range(PH):
        pltpu.sync_copy(dst_hbm.at[wid, pl.ds(ph * GP, GP)], idx_d)

        @pl.loop(0, GP)
        def _(g):
            pltpu.sync_copy(ones_v, acc.at[idx_d.at[g]], add=True)

    plsc.subcore_barrier()
    pltpu.sync_copy(acc.at[pl.ds(r0, RPW)], out_hbm.at[cid, pl.ds(r0, RPW)])


_SC_MESH = plsc.VectorSubcoreMesh(core_axis_name="c", subcore_axis_name="s")
_OUT_P = [jax.ShapeDtypeStruct((NC, NP, D), jnp.float32)]

_segsum = pl.kernel(
    _segsum_body, out_type=_OUT_P, mesh=_SC_MESH,
    scratch_types=[
        pltpu.VMEM((GP, K), jnp.int32),
        pltpu.VMEM((GP, K), jnp.int32),
        pltpu.VMEM((K,), jnp.int32),
        pltpu.VMEM((K,), jnp.int32),
        pltpu.VMEM((K, D), jnp.float32),
        pltpu.VMEM_SHARED((NP, D), jnp.float32),
    ])

_deg_sc = pl.kernel(
    _deg_body, out_type=_OUT_P, mesh=_SC_MESH,
    scratch_types=[
        pltpu.VMEM((GP, K), jnp.int32),
        pltpu.VMEM((K, D), jnp.float32),
        pltpu.VMEM_SHARED((NP, D), jnp.float32),
    ])


def _mm(a, b):
    return lax.dot_general(a, b, (((1,), (0,)), ((), ())),
                           precision=lax.Precision.HIGHEST,
                           preferred_element_type=jnp.float32)


def _acc_stats(i, y, stats_ref):
    @pl.when(i == 0)
    def _():
        stats_ref[...] = jnp.zeros((8, D), jnp.float32)
    stats_ref[0:1, :] += jnp.sum(y, axis=0, keepdims=True)
    stats_ref[1:2, :] += jnp.sum(y * y, axis=0, keepdims=True)


def _sage_a_body(p_ref, pdeg_ref, h_ref, wl_ref, bl_ref, wr_ref,
                 y_ref, stats_ref):
    i = pl.program_id(0)
    s = p_ref[0] + p_ref[1]
    deg = pdeg_ref[0, :, 0:1] + pdeg_ref[1, :, 0:1]
    mean = s / jnp.maximum(deg, 1.0)
    y = _mm(mean, wl_ref[...]) + bl_ref[...] + _mm(h_ref[...], wr_ref[...])
    y_ref[...] = y
    _acc_stats(i, y, stats_ref)


def _gin_a_body(p_ref, h_ref, w1_ref, b1_ref, w2_ref, b2_ref,
                y_ref, stats_ref):
    i = pl.program_id(0)
    hh = h_ref[...] + p_ref[0] + p_ref[1]
    a = jnp.maximum(_mm(hh, w1_ref[...]) + b1_ref[...], 0.0)
    y = _mm(a, w2_ref[...]) + b2_ref[...]
    y_ref[...] = y
    _acc_stats(i, y, stats_ref)


def _bn_b_body(y_ref, stats_ref, g_ref, b_ref, *rest):
    if len(rest) == 2:
        res_ref, o_ref = rest
    else:
        res_ref, (o_ref,) = None, rest
    m = stats_ref[0:1, :] / N
    v = stats_ref[1:2, :] / N - m * m
    z = (y_ref[...] - m) * lax.rsqrt(v + 1e-5) * g_ref[...] + b_ref[...]
    z = jnp.maximum(z, 0.0)
    if res_ref is not None:
        z = z + res_ref[...]
    o_ref[...] = z


def _final_body(p_ref, pdeg_ref, h_ref, wl_ref, bl_ref, wr_ref, o_ref):
    s = p_ref[0] + p_ref[1]
    deg = pdeg_ref[0, :, 0:1] + pdeg_ref[1, :, 0:1]
    mean = s / jnp.maximum(deg, 1.0)
    y = _mm(mean, wl_ref[...]) + bl_ref[...] + _mm(h_ref[...], wr_ref[...])
    mx = jnp.max(y, axis=1, keepdims=True)
    lse = jnp.log(jnp.sum(jnp.exp(y - mx), axis=1, keepdims=True)) + mx
    o_ref[...] = y - lse


_GRID = (N // MB,)
_BLK_P = pl.BlockSpec((NC, MB, D), lambda i: (0, i, 0))
_BLK_H = pl.BlockSpec((MB, D), lambda i: (i, 0))
_BLK_W = pl.BlockSpec((D, D), lambda i: (0, 0))
_BLK_B = pl.BlockSpec((1, D), lambda i: (0, 0))
_BLK_S = pl.BlockSpec((8, D), lambda i: (0, 0))
_OUT_Y = jax.ShapeDtypeStruct((N, D), jnp.float32)
_OUT_S = jax.ShapeDtypeStruct((8, D), jnp.float32)


def _sage_a(p, pdeg, h, wl, bl, wr):
    return pl.pallas_call(
        _sage_a_body, grid=_GRID,
        in_specs=[_BLK_P, _BLK_P, _BLK_H, _BLK_W, _BLK_B, _BLK_W],
        out_specs=[_BLK_H, _BLK_S],
        out_shape=[_OUT_Y, _OUT_S],
    )(p, pdeg, h, wl, bl, wr)


def _gin_a(p, h, w1, b1, w2, b2):
    return pl.pallas_call(
        _gin_a_body, grid=_GRID,
        in_specs=[_BLK_P, _BLK_H, _BLK_W, _BLK_B, _BLK_W, _BLK_B],
        out_specs=[_BLK_H, _BLK_S],
        out_shape=[_OUT_Y, _OUT_S],
    )(p, h, w1, b1, w2, b2)


def _bn_b(y, stats, g, b, res=None):
    in_specs = [_BLK_H, _BLK_S, _BLK_B, _BLK_B]
    args = [y, stats, g, b]
    if res is not None:
        in_specs.append(_BLK_H)
        args.append(res)
    return pl.pallas_call(
        _bn_b_body, grid=_GRID, in_specs=in_specs,
        out_specs=_BLK_H, out_shape=_OUT_Y,
    )(*args)


def _final_tc(p, pdeg, h, wl, bl, wr):
    return pl.pallas_call(
        _final_body, grid=_GRID,
        in_specs=[_BLK_P, _BLK_P, _BLK_H, _BLK_W, _BLK_B, _BLK_W],
        out_specs=_BLK_H, out_shape=_OUT_Y,
    )(p, pdeg, h, wl, bl, wr)


def kernel(x, edge_index, sage0_Wl, sage0_bl, sage0_Wr, gin1_W1, gin1_b1,
           gin1_W2, gin1_b2, sage2_Wl, sage2_bl, sage2_Wr, bn0_g, bn0_b,
           bn1_g, bn1_b, bn2_g, bn2_b, fin_Wl, fin_bl, fin_Wr):
    r = lambda v: v.reshape(1, D)
    zrow = jnp.zeros((RPW, D), jnp.float32)
    ones = jnp.ones((K, D), jnp.float32)
    src = jnp.pad(edge_index[0].reshape(NW, EPW), ((0, 0), (0, EPWP - EPW)),
                  constant_values=0).reshape(NW, G, K)
    dst = jnp.pad(edge_index[1].reshape(NW, EPW), ((0, 0), (0, EPWP - EPW)),
                  constant_values=NP - 1).reshape(NW, G, K)

    (pdeg,) = _deg_sc(dst, ones, zrow)
    (p,) = _segsum(x, src, dst, zrow)
    y, st = _sage_a(p, pdeg, x, sage0_Wl, r(sage0_bl), sage0_Wr)
    h1 = _bn_b(y, st, r(bn0_g), r(bn0_b))
    (p,) = _segsum(h1, src, dst, zrow)
    y, st = _gin_a(p, h1, gin1_W1, r(gin1_b1), gin1_W2, r(gin1_b2))
    h2 = _bn_b(y, st, r(bn1_g), r(bn1_b), res=h1)
    (p,) = _segsum(h2, src, dst, zrow)
    y, st = _sage_a(p, pdeg, h2, sage2_Wl, r(sage2_bl), sage2_Wr)
    h3 = _bn_b(y, st, r(bn2_g), r(bn2_b), res=h2)
    (p,) = _segsum(h3, src, dst, zrow)
    return _final_tc(p, pdeg, h3, fin_Wl, r(fin_bl), fin_Wr)


# speedup vs baseline: 1.9376x; 1.9376x over previous
import functools

import jax
import jax.numpy as jnp
from jax import lax
from jax.experimental import pallas as pl
from jax.experimental.pallas import tpu as pltpu
from jax.experimental.pallas import tpu_sc as plsc

N = 10000
E = 320000
D = 128
NP = 10240
NC = 2
NS = 16
NW = NC * NS
EPW = E // NW
K = 80
G = EPW // K
PH = 5
GP = G // PH
RPW = NP // NS
DW = 16
DC = 64
MB = 2000


def _segsum_body(h_hbm, src_hbm, dst_hbm, zrow_hbm, out_hbm,
                 idx_s, idx_d, idx1s, idx1d, rows_a, acc):
    cid = lax.axis_index("c")
    sid = lax.axis_index("s")
    wid = cid * NS + sid
    r0 = sid * RPW

    pltpu.sync_copy(zrow_hbm, acc.at[pl.ds(r0, RPW)])
    plsc.subcore_barrier()

    for ph in range(PH):
        base = wid * EPW + ph * (GP * K)
        pltpu.sync_copy(src_hbm.at[pl.ds(base, GP * K)], idx_s)
        pltpu.sync_copy(dst_hbm.at[pl.ds(base, GP * K)], idx_d)

        @pl.loop(0, GP)
        def _(g):
            for c in range(K // 16):
                idx1s[pl.ds(16 * c, 16)] = idx_s[pl.ds(g * K + 16 * c, 16)]
                idx1d[pl.ds(16 * c, 16)] = idx_d[pl.ds(g * K + 16 * c, 16)]
            pltpu.sync_copy(h_hbm.at[idx1s], rows_a)
            pltpu.sync_copy(rows_a, acc.at[idx1d], add=True)

    plsc.subcore_barrier()
    pltpu.sync_copy(acc.at[pl.ds(r0, RPW)], out_hbm.at[cid, pl.ds(r0, RPW)])


def _deg_body(dst_hbm, ones_hbm, zrow_hbm, out_hbm, idx_d, idx1d, ones_v,
              acc):
    cid = lax.axis_index("c")
    sid = lax.axis_index("s")
    wid = cid * NS + sid
    r0 = sid * RPW

    pltpu.sync_copy(zrow_hbm, acc.at[pl.ds(r0, RPW)])
    pltpu.sync_copy(ones_hbm, ones_v)
    plsc.subcore_barrier()

    for ph in range(PH):
        base = wid * EPW + ph * (GP * K)
        pltpu.sync_copy(dst_hbm.at[pl.ds(base, GP * K)], idx_d)

        @pl.loop(0, GP)
        def _(g):
            for c in range(K // 16):
                idx1d[pl.ds(16 * c, 16)] = idx_d[pl.ds(g * K + 16 * c, 16)]
            pltpu.sync_copy(ones_v, acc.at[idx1d], add=True)

    plsc.subcore_barrier()
    pltpu.sync_copy(acc.at[pl.ds(r0, RPW)], out_hbm.at[cid, pl.ds(r0, RPW)])


_SC_MESH = plsc.VectorSubcoreMesh(core_axis_name="c", subcore_axis_name="s")
_OUT_P = [jax.ShapeDtypeStruct((NC, NP, D), jnp.float32)]

_segsum = pl.kernel(
    _segsum_body, out_type=_OUT_P, mesh=_SC_MESH,
    scratch_types=[
        pltpu.VMEM((GP * K,), jnp.int32),
        pltpu.VMEM((GP * K,), jnp.int32),
        pltpu.VMEM((K,), jnp.int32),
        pltpu.VMEM((K,), jnp.int32),
        pltpu.VMEM((K, D), jnp.float32),
        pltpu.VMEM_SHARED((NP, D), jnp.float32),
    ])

_deg_sc = pl.kernel(
    _deg_body, out_type=_OUT_P, mesh=_SC_MESH,
    scratch_types=[
        pltpu.VMEM((GP * K,), jnp.int32),
        pltpu.VMEM((K,), jnp.int32),
        pltpu.VMEM((K, D), jnp.float32),
        pltpu.VMEM_SHARED((NP, D), jnp.float32),
    ])


def _mm(a, b):
    return lax.dot_general(a, b, (((1,), (0,)), ((), ())),
                           precision=lax.Precision.HIGHEST,
                           preferred_element_type=jnp.float32)


def _acc_stats(i, y, stats_ref):
    @pl.when(i == 0)
    def _():
        stats_ref[...] = jnp.zeros((8, D), jnp.float32)
    stats_ref[0:1, :] += jnp.sum(y, axis=0, keepdims=True)
    stats_ref[1:2, :] += jnp.sum(y * y, axis=0, keepdims=True)


def _sage_a_body(p_ref, pdeg_ref, h_ref, wl_ref, bl_ref, wr_ref,
                 y_ref, stats_ref):
    i = pl.program_id(0)
    s = p_ref[0] + p_ref[1]
    deg = pdeg_ref[0, :, 0:1] + pdeg_ref[1, :, 0:1]
    mean = s / jnp.maximum(deg, 1.0)
    y = _mm(mean, wl_ref[...]) + bl_ref[...] + _mm(h_ref[...], wr_ref[...])
    y_ref[...] = y
    _acc_stats(i, y, stats_ref)


def _gin_a_body(p_ref, h_ref, w1_ref, b1_ref, w2_ref, b2_ref,
                y_ref, stats_ref):
    i = pl.program_id(0)
    hh = h_ref[...] + p_ref[0] + p_ref[1]
    a = jnp.maximum(_mm(hh, w1_ref[...]) + b1_ref[...], 0.0)
    y = _mm(a, w2_ref[...]) + b2_ref[...]
    y_ref[...] = y
    _acc_stats(i, y, stats_ref)


def _bn_b_body(y_ref, stats_ref, g_ref, b_ref, *rest):
    if len(rest) == 2:
        res_ref, o_ref = rest
    else:
        res_ref, (o_ref,) = None, rest
    m = stats_ref[0:1, :] / N
    v = stats_ref[1:2, :] / N - m * m
    z = (y_ref[...] - m) * lax.rsqrt(v + 1e-5) * g_ref[...] + b_ref[...]
    z = jnp.maximum(z, 0.0)
    if res_ref is not None:
        z = z + res_ref[...]
    o_ref[...] = z


def _final_body(p_ref, pdeg_ref, h_ref, wl_ref, bl_ref, wr_ref, o_ref):
    s = p_ref[0] + p_ref[1]
    deg = pdeg_ref[0, :, 0:1] + pdeg_ref[1, :, 0:1]
    mean = s / jnp.maximum(deg, 1.0)
    y = _mm(mean, wl_ref[...]) + bl_ref[...] + _mm(h_ref[...], wr_ref[...])
    mx = jnp.max(y, axis=1, keepdims=True)
    lse = jnp.log(jnp.sum(jnp.exp(y - mx), axis=1, keepdims=True)) + mx
    o_ref[...] = y - lse


_GRID = (N // MB,)
_BLK_P = pl.BlockSpec((NC, MB, D), lambda i: (0, i, 0))
_BLK_H = pl.BlockSpec((MB, D), lambda i: (i, 0))
_BLK_W = pl.BlockSpec((D, D), lambda i: (0, 0))
_BLK_B = pl.BlockSpec((1, D), lambda i: (0, 0))
_BLK_S = pl.BlockSpec((8, D), lambda i: (0, 0))
_OUT_Y = jax.ShapeDtypeStruct((N, D), jnp.float32)
_OUT_S = jax.ShapeDtypeStruct((8, D), jnp.float32)


def _sage_a(p, pdeg, h, wl, bl, wr):
    return pl.pallas_call(
        _sage_a_body, grid=_GRID,
        in_specs=[_BLK_P, _BLK_P, _BLK_H, _BLK_W, _BLK_B, _BLK_W],
        out_specs=[_BLK_H, _BLK_S],
        out_shape=[_OUT_Y, _OUT_S],
    )(p, pdeg, h, wl, bl, wr)


def _gin_a(p, h, w1, b1, w2, b2):
    return pl.pallas_call(
        _gin_a_body, grid=_GRID,
        in_specs=[_BLK_P, _BLK_H, _BLK_W, _BLK_B, _BLK_W, _BLK_B],
        out_specs=[_BLK_H, _BLK_S],
        out_shape=[_OUT_Y, _OUT_S],
    )(p, h, w1, b1, w2, b2)


def _bn_b(y, stats, g, b, res=None):
    in_specs = [_BLK_H, _BLK_S, _BLK_B, _BLK_B]
    args = [y, stats, g, b]
    if res is not None:
        in_specs.append(_BLK_H)
        args.append(res)
    return pl.pallas_call(
        _bn_b_body, grid=_GRID, in_specs=in_specs,
        out_specs=_BLK_H, out_shape=_OUT_Y,
    )(*args)


def _final_tc(p, pdeg, h, wl, bl, wr):
    return pl.pallas_call(
        _final_body, grid=_GRID,
        in_specs=[_BLK_P, _BLK_P, _BLK_H, _BLK_W, _BLK_B, _BLK_W],
        out_specs=_BLK_H, out_shape=_OUT_Y,
    )(p, pdeg, h, wl, bl, wr)


def kernel(x, edge_index, sage0_Wl, sage0_bl, sage0_Wr, gin1_W1, gin1_b1,
           gin1_W2, gin1_b2, sage2_Wl, sage2_bl, sage2_Wr, bn0_g, bn0_b,
           bn1_g, bn1_b, bn2_g, bn2_b, fin_Wl, fin_bl, fin_Wr):
    r = lambda v: v.reshape(1, D)
    zrow = jnp.zeros((RPW, D), jnp.float32)
    ones = jnp.ones((K, D), jnp.float32)
    src = edge_index[0]
    dst = edge_index[1]

    (pdeg,) = _deg_sc(dst, ones, zrow)
    (p,) = _segsum(x, src, dst, zrow)
    y, st = _sage_a(p, pdeg, x, sage0_Wl, r(sage0_bl), sage0_Wr)
    h1 = _bn_b(y, st, r(bn0_g), r(bn0_b))
    (p,) = _segsum(h1, src, dst, zrow)
    y, st = _gin_a(p, h1, gin1_W1, r(gin1_b1), gin1_W2, r(gin1_b2))
    h2 = _bn_b(y, st, r(bn1_g), r(bn1_b), res=h1)
    (p,) = _segsum(h2, src, dst, zrow)
    y, st = _sage_a(p, pdeg, h2, sage2_Wl, r(sage2_bl), sage2_Wr)
    h3 = _bn_b(y, st, r(bn2_g), r(bn2_b), res=h2)
    (p,) = _segsum(h3, src, dst, zrow)
    return _final_tc(p, pdeg, h3, fin_Wl, r(fin_bl), fin_Wr)


# speedup vs baseline: 2.3685x; 1.2224x over previous
import functools

import jax
import jax.numpy as jnp
from jax import lax
from jax.experimental import pallas as pl
from jax.experimental.pallas import tpu as pltpu
from jax.experimental.pallas import tpu_sc as plsc

N = 10000
E = 320000
D = 128
NP = 10240
NC = 2
NS = 16
NW = NC * NS
EPW = E // NW
K = 80
G = EPW // K
PH = 5
GP = G // PH
RPW = NP // NS
DW = 16
DC = 64
MB = 2000


def _segsum_body(h_hbm, src_hbm, dst_hbm, zrow_hbm, out_hbm,
                 idx_s, idx_d, i1sa, i1da, i1sb, i1db, rows_a, rows_b,
                 sem_a, sem_b, acc):
    cid = lax.axis_index("c")
    sid = lax.axis_index("s")
    wid = cid * NS + sid
    r0 = sid * RPW

    pltpu.sync_copy(zrow_hbm, acc.at[pl.ds(r0, RPW)])
    plsc.subcore_barrier()

    def copyidx(g, i1s, i1d):
        for c in range(K // 16):
            i1s[pl.ds(16 * c, 16)] = idx_s[pl.ds(g * K + 16 * c, 16)]
            i1d[pl.ds(16 * c, 16)] = idx_d[pl.ds(g * K + 16 * c, 16)]

    def startg(i1s, rows, sem):
        pltpu.make_async_copy(h_hbm.at[i1s], rows, sem).start()

    def waitg(i1s, rows, sem):
        pltpu.make_async_copy(h_hbm.at[i1s], rows, sem).wait()

    def scat(i1d, rows):
        pltpu.sync_copy(rows, acc.at[i1d], add=True)

    for ph in range(PH):
        base = wid * EPW + ph * (GP * K)
        pltpu.sync_copy(src_hbm.at[pl.ds(base, GP * K)], idx_s)
        pltpu.sync_copy(dst_hbm.at[pl.ds(base, GP * K)], idx_d)

        copyidx(0, i1sa, i1da)
        startg(i1sa, rows_a, sem_a)

        @pl.loop(0, GP, step=2)
        def _(g):
            waitg(i1sa, rows_a, sem_a)

            @pl.when(g + 1 < GP)
            def _():
                copyidx(g + 1, i1sb, i1db)
                startg(i1sb, rows_b, sem_b)

            scat(i1da, rows_a)

            @pl.when(g + 1 < GP)
            def _():
                waitg(i1sb, rows_b, sem_b)

                @pl.when(g + 2 < GP)
                def _():
                    copyidx(g + 2, i1sa, i1da)
                    startg(i1sa, rows_a, sem_a)

                scat(i1db, rows_b)

    plsc.subcore_barrier()
    pltpu.sync_copy(acc.at[pl.ds(r0, RPW)], out_hbm.at[cid, pl.ds(r0, RPW)])


def _deg_body(dst_hbm, ones_hbm, zrow_hbm, out_hbm, idx_d, idx1d, ones_v,
              acc):
    cid = lax.axis_index("c")
    sid = lax.axis_index("s")
    wid = cid * NS + sid
    r0 = sid * RPW

    pltpu.sync_copy(zrow_hbm, acc.at[pl.ds(r0, RPW)])
    pltpu.sync_copy(ones_hbm, ones_v)
    plsc.subcore_barrier()

    for ph in range(PH):
        base = wid * EPW + ph * (GP * K)
        pltpu.sync_copy(dst_hbm.at[pl.ds(base, GP * K)], idx_d)

        @pl.loop(0, GP)
        def _(g):
            for c in range(K // 16):
                idx1d[pl.ds(16 * c, 16)] = idx_d[pl.ds(g * K + 16 * c, 16)]
            pltpu.sync_copy(ones_v, acc.at[idx1d], add=True)

    plsc.subcore_barrier()
    pltpu.sync_copy(acc.at[pl.ds(r0, RPW)], out_hbm.at[cid, pl.ds(r0, RPW)])


_SC_MESH = plsc.VectorSubcoreMesh(core_axis_name="c", subcore_axis_name="s")
_OUT_P = [jax.ShapeDtypeStruct((NC, NP, D), jnp.float32)]

_segsum = pl.kernel(
    _segsum_body, out_type=_OUT_P, mesh=_SC_MESH,
    scratch_types=[
        pltpu.VMEM((GP * K,), jnp.int32),
        pltpu.VMEM((GP * K,), jnp.int32),
        pltpu.VMEM((K,), jnp.int32),
        pltpu.VMEM((K,), jnp.int32),
        pltpu.VMEM((K,), jnp.int32),
        pltpu.VMEM((K,), jnp.int32),
        pltpu.VMEM((K, D), jnp.float32),
        pltpu.VMEM((K, D), jnp.float32),
        pltpu.SemaphoreType.DMA,
        pltpu.SemaphoreType.DMA,
        pltpu.VMEM_SHARED((NP, D), jnp.float32),
    ])

_deg_sc = pl.kernel(
    _deg_body, out_type=_OUT_P, mesh=_SC_MESH,
    scratch_types=[
        pltpu.VMEM((GP * K,), jnp.int32),
        pltpu.VMEM((K,), jnp.int32),
        pltpu.VMEM((K, D), jnp.float32),
        pltpu.VMEM_SHARED((NP, D), jnp.float32),
    ])


def _mm(a, b):
    return lax.dot_general(a, b, (((1,), (0,)), ((), ())),
                           precision=lax.Precision.HIGHEST,
                           preferred_element_type=jnp.float32)


def _acc_stats(i, y, stats_ref):
    @pl.when(i == 0)
    def _():
        stats_ref[...] = jnp.zeros((8, D), jnp.float32)
    stats_ref[0:1, :] += jnp.sum(y, axis=0, keepdims=True)
    stats_ref[1:2, :] += jnp.sum(y * y, axis=0, keepdims=True)


def _sage_a_body(p_ref, pdeg_ref, h_ref, wl_ref, bl_ref, wr_ref,
                 y_ref, stats_ref):
    i = pl.program_id(0)
    s = p_ref[0] + p_ref[1]
    deg = pdeg_ref[0, :, 0:1] + pdeg_ref[1, :, 0:1]
    mean = s / jnp.maximum(deg, 1.0)
    y = _mm(mean, wl_ref[...]) + bl_ref[...] + _mm(h_ref[...], wr_ref[...])
    y_ref[...] = y
    _acc_stats(i, y, stats_ref)


def _gin_a_body(p_ref, h_ref, w1_ref, b1_ref, w2_ref, b2_ref,
                y_ref, stats_ref):
    i = pl.program_id(0)
    hh = h_ref[...] + p_ref[0] + p_ref[1]
    a = jnp.maximum(_mm(hh, w1_ref[...]) + b1_ref[...], 0.0)
    y = _mm(a, w2_ref[...]) + b2_ref[...]
    y_ref[...] = y
    _acc_stats(i, y, stats_ref)


def _bn_b_body(y_ref, stats_ref, g_ref, b_ref, *rest):
    if len(rest) == 2:
        res_ref, o_ref = rest
    else:
        res_ref, (o_ref,) = None, rest
    m = stats_ref[0:1, :] / N
    v = stats_ref[1:2, :] / N - m * m
    z = (y_ref[...] - m) * lax.rsqrt(v + 1e-5) * g_ref[...] + b_ref[...]
    z = jnp.maximum(z, 0.0)
    if res_ref is not None:
        z = z + res_ref[...]
    o_ref[...] = z


def _final_body(p_ref, pdeg_ref, h_ref, wl_ref, bl_ref, wr_ref, o_ref):
    s = p_ref[0] + p_ref[1]
    deg = pdeg_ref[0, :, 0:1] + pdeg_ref[1, :, 0:1]
    mean = s / jnp.maximum(deg, 1.0)
    y = _mm(mean, wl_ref[...]) + bl_ref[...] + _mm(h_ref[...], wr_ref[...])
    mx = jnp.max(y, axis=1, keepdims=True)
    lse = jnp.log(jnp.sum(jnp.exp(y - mx), axis=1, keepdims=True)) + mx
    o_ref[...] = y - lse


_GRID = (N // MB,)
_BLK_P = pl.BlockSpec((NC, MB, D), lambda i: (0, i, 0))
_BLK_H = pl.BlockSpec((MB, D), lambda i: (i, 0))
_BLK_W = pl.BlockSpec((D, D), lambda i: (0, 0))
_BLK_B = pl.BlockSpec((1, D), lambda i: (0, 0))
_BLK_S = pl.BlockSpec((8, D), lambda i: (0, 0))
_OUT_Y = jax.ShapeDtypeStruct((N, D), jnp.float32)
_OUT_S = jax.ShapeDtypeStruct((8, D), jnp.float32)


def _sage_a(p, pdeg, h, wl, bl, wr):
    return pl.pallas_call(
        _sage_a_body, grid=_GRID,
        in_specs=[_BLK_P, _BLK_P, _BLK_H, _BLK_W, _BLK_B, _BLK_W],
        out_specs=[_BLK_H, _BLK_S],
        out_shape=[_OUT_Y, _OUT_S],
    )(p, pdeg, h, wl, bl, wr)


def _gin_a(p, h, w1, b1, w2, b2):
    return pl.pallas_call(
        _gin_a_body, grid=_GRID,
        in_specs=[_BLK_P, _BLK_H, _BLK_W, _BLK_B, _BLK_W, _BLK_B],
        out_specs=[_BLK_H, _BLK_S],
        out_shape=[_OUT_Y, _OUT_S],
    )(p, h, w1, b1, w2, b2)


def _bn_b(y, stats, g, b, res=None):
    in_specs = [_BLK_H, _BLK_S, _BLK_B, _BLK_B]
    args = [y, stats, g, b]
    if res is not None:
        in_specs.append(_BLK_H)
        args.append(res)
    return pl.pallas_call(
        _bn_b_body, grid=_GRID, in_specs=in_specs,
        out_specs=_BLK_H, out_shape=_OUT_Y,
    )(*args)


def _final_tc(p, pdeg, h, wl, bl, wr):
    return pl.pallas_call(
        _final_body, grid=_GRID,
        in_specs=[_BLK_P, _BLK_P, _BLK_H, _BLK_W, _BLK_B, _BLK_W],
        out_specs=_BLK_H, out_shape=_OUT_Y,
    )(p, pdeg, h, wl, bl, wr)


def kernel(x, edge_index, sage0_Wl, sage0_bl, sage0_Wr, gin1_W1, gin1_b1,
           gin1_W2, gin1_b2, sage2_Wl, sage2_bl, sage2_Wr, bn0_g, bn0_b,
           bn1_g, bn1_b, bn2_g, bn2_b, fin_Wl, fin_bl, fin_Wr):
    r = lambda v: v.reshape(1, D)
    zrow = jnp.zeros((RPW, D), jnp.float32)
    ones = jnp.ones((K, D), jnp.float32)
    src = edge_index[0]
    dst = edge_index[1]

    (pdeg,) = _deg_sc(dst, ones, zrow)
    (p,) = _segsum(x, src, dst, zrow)
    y, st = _sage_a(p, pdeg, x, sage0_Wl, r(sage0_bl), sage0_Wr)
    h1 = _bn_b(y, st, r(bn0_g), r(bn0_b))
    (p,) = _segsum(h1, src, dst, zrow)
    y, st = _gin_a(p, h1, gin1_W1, r(gin1_b1), gin1_W2, r(gin1_b2))
    h2 = _bn_b(y, st, r(bn1_g), r(bn1_b), res=h1)
    (p,) = _segsum(h2, src, dst, zrow)
    y, st = _sage_a(p, pdeg, h2, sage2_Wl, r(sage2_bl), sage2_Wr)
    h3 = _bn_b(y, st, r(bn2_g), r(bn2_b), res=h2)
    (p,) = _segsum(h3, src, dst, zrow)
    return _final_tc(p, pdeg, h3, fin_Wl, r(fin_bl), fin_Wr)


# speedup vs baseline: 3.2060x; 1.3536x over previous
import functools

import jax
import jax.numpy as jnp
from jax import lax
from jax.experimental import pallas as pl
from jax.experimental.pallas import tpu as pltpu
from jax.experimental.pallas import tpu_sc as plsc

N = 10000
E = 320000
D = 128
NP = 10240
NC = 2
NS = 16
NW = NC * NS
EPW = E // NW
K = 80
G = EPW // K
PH = 5
GP = G // PH
RPW = NP // NS
DW = 16
DC = 64
MB = 2000


def _segsum_body(h_hbm, src_hbm, dst_hbm, zrow_hbm, out_hbm,
                 idx_s, idx_d, i1sa, i1da, i1sb, i1db, i1sc, i1dc,
                 rows_a, rows_b, rows_c, sem_a, sem_b, sem_c, acc):
    cid = lax.axis_index("c")
    sid = lax.axis_index("s")
    wid = cid * NS + sid
    r0 = sid * RPW

    pltpu.sync_copy(zrow_hbm, acc.at[pl.ds(r0, RPW)])
    plsc.subcore_barrier()

    def copyidx(g, i1s, i1d):
        for c in range(K // 16):
            i1s[pl.ds(16 * c, 16)] = idx_s[pl.ds(g * K + 16 * c, 16)]
            i1d[pl.ds(16 * c, 16)] = idx_d[pl.ds(g * K + 16 * c, 16)]

    def startg(i1s, rows, sem):
        pltpu.make_async_copy(h_hbm.at[i1s], rows, sem).start()

    def waitg(i1s, rows, sem):
        pltpu.make_async_copy(h_hbm.at[i1s], rows, sem).wait()

    def scat(i1d, rows):
        pltpu.sync_copy(rows, acc.at[i1d], add=True)

    for ph in range(PH):
        base = wid * EPW + ph * (GP * K)
        pltpu.sync_copy(src_hbm.at[pl.ds(base, GP * K)], idx_s)
        pltpu.sync_copy(dst_hbm.at[pl.ds(base, GP * K)], idx_d)

        bufs = [(i1sa, i1da, rows_a, sem_a),
                (i1sb, i1db, rows_b, sem_b),
                (i1sc, i1dc, rows_c, sem_c)]
        copyidx(0, i1sa, i1da)
        startg(i1sa, rows_a, sem_a)
        copyidx(1, i1sb, i1db)
        startg(i1sb, rows_b, sem_b)

        @pl.loop(0, GP, step=3)
        def _(g0):
            for j in range(3):
                cs, cd, cr, csem = bufs[j]
                ns, nd, nr, nsem = bufs[(j + 2) % 3]
                g = g0 + j

                @pl.when(g < GP)
                def _():
                    waitg(cs, cr, csem)

                    @pl.when(g + 2 < GP)
                    def _():
                        copyidx(g + 2, ns, nd)
                        startg(ns, nr, nsem)

                    scat(cd, cr)

    plsc.subcore_barrier()
    pltpu.sync_copy(acc.at[pl.ds(r0, RPW)], out_hbm.at[cid, pl.ds(r0, RPW)])


def _deg_body(dst_hbm, ones_hbm, zrow_hbm, out_hbm, idx_d, idx1d, ones_v,
              acc):
    cid = lax.axis_index("c")
    sid = lax.axis_index("s")
    wid = cid * NS + sid
    r0 = sid * RPW

    pltpu.sync_copy(zrow_hbm, acc.at[pl.ds(r0, RPW)])
    pltpu.sync_copy(ones_hbm, ones_v)
    plsc.subcore_barrier()

    for ph in range(PH):
        base = wid * EPW + ph * (GP * K)
        pltpu.sync_copy(dst_hbm.at[pl.ds(base, GP * K)], idx_d)

        @pl.loop(0, GP)
        def _(g):
            for c in range(K // 16):
                idx1d[pl.ds(16 * c, 16)] = idx_d[pl.ds(g * K + 16 * c, 16)]
            pltpu.sync_copy(ones_v, acc.at[idx1d], add=True)

    plsc.subcore_barrier()
    pltpu.sync_copy(acc.at[pl.ds(r0, RPW)], out_hbm.at[cid, pl.ds(r0, RPW)])


_SC_MESH = plsc.VectorSubcoreMesh(core_axis_name="c", subcore_axis_name="s")
_OUT_P = [jax.ShapeDtypeStruct((NC, NP, D), jnp.float32)]

_segsum = pl.kernel(
    _segsum_body, out_type=_OUT_P, mesh=_SC_MESH,
    scratch_types=[
        pltpu.VMEM((GP * K,), jnp.int32),
        pltpu.VMEM((GP * K,), jnp.int32),
        pltpu.VMEM((K,), jnp.int32),
        pltpu.VMEM((K,), jnp.int32),
        pltpu.VMEM((K,), jnp.int32),
        pltpu.VMEM((K,), jnp.int32),
        pltpu.VMEM((K,), jnp.int32),
        pltpu.VMEM((K,), jnp.int32),
        pltpu.VMEM((K, D), jnp.float32),
        pltpu.VMEM((K, D), jnp.float32),
        pltpu.VMEM((K, D), jnp.float32),
        pltpu.SemaphoreType.DMA,
        pltpu.SemaphoreType.DMA,
        pltpu.SemaphoreType.DMA,
        pltpu.VMEM_SHARED((NP, D), jnp.float32),
    ])

_deg_sc = pl.kernel(
    _deg_body, out_type=_OUT_P, mesh=_SC_MESH,
    scratch_types=[
        pltpu.VMEM((GP * K,), jnp.int32),
        pltpu.VMEM((K,), jnp.int32),
        pltpu.VMEM((K, D), jnp.float32),
        pltpu.VMEM_SHARED((NP, D), jnp.float32),
    ])


def _mm(a, b):
    return lax.dot_general(a, b, (((1,), (0,)), ((), ())),
                           precision=lax.Precision.HIGHEST,
                           preferred_element_type=jnp.float32)


def _acc_stats(i, y, stats_ref):
    @pl.when(i == 0)
    def _():
        stats_ref[...] = jnp.zeros((8, D), jnp.float32)
    stats_ref[0:1, :] += jnp.sum(y, axis=0, keepdims=True)
    stats_ref[1:2, :] += jnp.sum(y * y, axis=0, keepdims=True)


def _sage_a_body(p_ref, pdeg_ref, h_ref, wl_ref, bl_ref, wr_ref,
                 y_ref, stats_ref):
    i = pl.program_id(0)
    s = p_ref[0] + p_ref[1]
    deg = pdeg_ref[0, :, 0:1] + pdeg_ref[1, :, 0:1]
    mean = s / jnp.maximum(deg, 1.0)
    y = _mm(mean, wl_ref[...]) + bl_ref[...] + _mm(h_ref[...], wr_ref[...])
    y_ref[...] = y
    _acc_stats(i, y, stats_ref)


def _gin_a_body(p_ref, h_ref, w1_ref, b1_ref, w2_ref, b2_ref,
                y_ref, stats_ref):
    i = pl.program_id(0)
    hh = h_ref[...] + p_ref[0] + p_ref[1]
    a = jnp.maximum(_mm(hh, w1_ref[...]) + b1_ref[...], 0.0)
    y = _mm(a, w2_ref[...]) + b2_ref[...]
    y_ref[...] = y
    _acc_stats(i, y, stats_ref)


def _bn_b_body(y_ref, stats_ref, g_ref, b_ref, *rest):
    if len(rest) == 2:
        res_ref, o_ref = rest
    else:
        res_ref, (o_ref,) = None, rest
    m = stats_ref[0:1, :] / N
    v = stats_ref[1:2, :] / N - m * m
    z = (y_ref[...] - m) * lax.rsqrt(v + 1e-5) * g_ref[...] + b_ref[...]
    z = jnp.maximum(z, 0.0)
    if res_ref is not None:
        z = z + res_ref[...]
    o_ref[...] = z


def _final_body(p_ref, pdeg_ref, h_ref, wl_ref, bl_ref, wr_ref, o_ref):
    s = p_ref[0] + p_ref[1]
    deg = pdeg_ref[0, :, 0:1] + pdeg_ref[1, :, 0:1]
    mean = s / jnp.maximum(deg, 1.0)
    y = _mm(mean, wl_ref[...]) + bl_ref[...] + _mm(h_ref[...], wr_ref[...])
    mx = jnp.max(y, axis=1, keepdims=True)
    lse = jnp.log(jnp.sum(jnp.exp(y - mx), axis=1, keepdims=True)) + mx
    o_ref[...] = y - lse


_GRID = (N // MB,)
_BLK_P = pl.BlockSpec((NC, MB, D), lambda i: (0, i, 0))
_BLK_H = pl.BlockSpec((MB, D), lambda i: (i, 0))
_BLK_W = pl.BlockSpec((D, D), lambda i: (0, 0))
_BLK_B = pl.BlockSpec((1, D), lambda i: (0, 0))
_BLK_S = pl.BlockSpec((8, D), lambda i: (0, 0))
_OUT_Y = jax.ShapeDtypeStruct((N, D), jnp.float32)
_OUT_S = jax.ShapeDtypeStruct((8, D), jnp.float32)


def _sage_a(p, pdeg, h, wl, bl, wr):
    return pl.pallas_call(
        _sage_a_body, grid=_GRID,
        in_specs=[_BLK_P, _BLK_P, _BLK_H, _BLK_W, _BLK_B, _BLK_W],
        out_specs=[_BLK_H, _BLK_S],
        out_shape=[_OUT_Y, _OUT_S],
    )(p, pdeg, h, wl, bl, wr)


def _gin_a(p, h, w1, b1, w2, b2):
    return pl.pallas_call(
        _gin_a_body, grid=_GRID,
        in_specs=[_BLK_P, _BLK_H, _BLK_W, _BLK_B, _BLK_W, _BLK_B],
        out_specs=[_BLK_H, _BLK_S],
        out_shape=[_OUT_Y, _OUT_S],
    )(p, h, w1, b1, w2, b2)


def _bn_b(y, stats, g, b, res=None):
    in_specs = [_BLK_H, _BLK_S, _BLK_B, _BLK_B]
    args = [y, stats, g, b]
    if res is not None:
        in_specs.append(_BLK_H)
        args.append(res)
    return pl.pallas_call(
        _bn_b_body, grid=_GRID, in_specs=in_specs,
        out_specs=_BLK_H, out_shape=_OUT_Y,
    )(*args)


def _final_tc(p, pdeg, h, wl, bl, wr):
    return pl.pallas_call(
        _final_body, grid=_GRID,
        in_specs=[_BLK_P, _BLK_P, _BLK_H, _BLK_W, _BLK_B, _BLK_W],
        out_specs=_BLK_H, out_shape=_OUT_Y,
    )(p, pdeg, h, wl, bl, wr)


def kernel(x, edge_index, sage0_Wl, sage0_bl, sage0_Wr, gin1_W1, gin1_b1,
           gin1_W2, gin1_b2, sage2_Wl, sage2_bl, sage2_Wr, bn0_g, bn0_b,
           bn1_g, bn1_b, bn2_g, bn2_b, fin_Wl, fin_bl, fin_Wr):
    r = lambda v: v.reshape(1, D)
    zrow = jnp.zeros((RPW, D), jnp.float32)
    ones = jnp.ones((K, D), jnp.float32)
    src = edge_index[0]
    dst = edge_index[1]

    (pdeg,) = _deg_sc(dst, ones, zrow)
    (p,) = _segsum(x, src, dst, zrow)
    y, st = _sage_a(p, pdeg, x, sage0_Wl, r(sage0_bl), sage0_Wr)
    h1 = _bn_b(y, st, r(bn0_g), r(bn0_b))
    (p,) = _segsum(h1, src, dst, zrow)
    y, st = _gin_a(p, h1, gin1_W1, r(gin1_b1), gin1_W2, r(gin1_b2))
    h2 = _bn_b(y, st, r(bn1_g), r(bn1_b), res=h1)
    (p,) = _segsum(h2, src, dst, zrow)
    y, st = _sage_a(p, pdeg, h2, sage2_Wl, r(sage2_bl), sage2_Wr)
    h3 = _bn_b(y, st, r(bn2_g), r(bn2_b), res=h2)
    (p,) = _segsum(h3, src, dst, zrow)
    return _final_tc(p, pdeg, h3, fin_Wl, r(fin_bl), fin_Wr)


# speedup vs baseline: 3.2782x; 1.0225x over previous
import functools

import jax
import jax.numpy as jnp
from jax import lax
from jax.experimental import pallas as pl
from jax.experimental.pallas import tpu as pltpu
from jax.experimental.pallas import tpu_sc as plsc

N = 10000
E = 320000
D = 128
NP = 10240
NC = 2
NS = 16
NW = NC * NS
EPW = E // NW
K = 80
G = EPW // K
PH = 5
GP = G // PH
RPW = NP // NS
DW = 16
DC = 64
MB = 2000


def _segsum_body(h_hbm, src_hbm, dst_hbm, zrow_hbm, out_hbm,
                 idx_s, idx_d, i1sa, i1da, i1sb, i1db, i1sc, i1dc,
                 i1sd, i1dd, rows_a, rows_b, rows_c, rows_d,
                 sem_a, sem_b, sem_c, sem_d, acc):
    cid = lax.axis_index("c")
    sid = lax.axis_index("s")
    wid = cid * NS + sid
    r0 = sid * RPW

    pltpu.sync_copy(zrow_hbm, acc.at[pl.ds(r0, RPW)])
    plsc.subcore_barrier()

    def copyidx(g, i1s, i1d):
        for c in range(K // 16):
            i1s[pl.ds(16 * c, 16)] = idx_s[pl.ds(g * K + 16 * c, 16)]
            i1d[pl.ds(16 * c, 16)] = idx_d[pl.ds(g * K + 16 * c, 16)]

    def startg(i1s, rows, sem):
        pltpu.make_async_copy(h_hbm.at[i1s], rows, sem).start()

    def waitg(i1s, rows, sem):
        pltpu.make_async_copy(h_hbm.at[i1s], rows, sem).wait()

    def scat(i1d, rows):
        pltpu.sync_copy(rows, acc.at[i1d], add=True)

    for ph in range(PH):
        base = wid * EPW + ph * (GP * K)
        pltpu.sync_copy(src_hbm.at[pl.ds(base, GP * K)], idx_s)
        pltpu.sync_copy(dst_hbm.at[pl.ds(base, GP * K)], idx_d)

        bufs = [(i1sa, i1da, rows_a, sem_a),
                (i1sb, i1db, rows_b, sem_b),
                (i1sc, i1dc, rows_c, sem_c),
                (i1sd, i1dd, rows_d, sem_d)]
        for j in range(3):
            copyidx(j, bufs[j][0], bufs[j][1])
            startg(bufs[j][0], bufs[j][2], bufs[j][3])

        @pl.loop(0, GP, step=4)
        def _(g0):
            for j in range(4):
                cs, cd, cr, csem = bufs[j]
                ns, nd, nr, nsem = bufs[(j + 3) % 4]
                g = g0 + j

                @pl.when(g < GP)
                def _():
                    waitg(cs, cr, csem)

                    @pl.when(g + 3 < GP)
                    def _():
                        copyidx(g + 3, ns, nd)
                        startg(ns, nr, nsem)

                    scat(cd, cr)

    plsc.subcore_barrier()
    pltpu.sync_copy(acc.at[pl.ds(r0, RPW)], out_hbm.at[cid, pl.ds(r0, RPW)])


def _deg_body(dst_hbm, ones_hbm, zrow_hbm, out_hbm, idx_d, idx1d, ones_v,
              acc):
    cid = lax.axis_index("c")
    sid = lax.axis_index("s")
    wid = cid * NS + sid
    r0 = sid * RPW

    pltpu.sync_copy(zrow_hbm, acc.at[pl.ds(r0, RPW)])
    pltpu.sync_copy(ones_hbm, ones_v)
    plsc.subcore_barrier()

    for ph in range(PH):
        base = wid * EPW + ph * (GP * K)
        pltpu.sync_copy(dst_hbm.at[pl.ds(base, GP * K)], idx_d)

        @pl.loop(0, GP)
        def _(g):
            for c in range(K // 16):
                idx1d[pl.ds(16 * c, 16)] = idx_d[pl.ds(g * K + 16 * c, 16)]
            pltpu.sync_copy(ones_v, acc.at[idx1d], add=True)

    plsc.subcore_barrier()
    pltpu.sync_copy(acc.at[pl.ds(r0, RPW)], out_hbm.at[cid, pl.ds(r0, RPW)])


_SC_MESH = plsc.VectorSubcoreMesh(core_axis_name="c", subcore_axis_name="s")
_OUT_P = [jax.ShapeDtypeStruct((NC, NP, D), jnp.float32)]

_segsum = pl.kernel(
    _segsum_body, out_type=_OUT_P, mesh=_SC_MESH,
    scratch_types=[
        pltpu.VMEM((GP * K,), jnp.int32),
        pltpu.VMEM((GP * K,), jnp.int32),
        pltpu.VMEM((K,), jnp.int32),
        pltpu.VMEM((K,), jnp.int32),
        pltpu.VMEM((K,), jnp.int32),
        pltpu.VMEM((K,), jnp.int32),
        pltpu.VMEM((K,), jnp.int32),
        pltpu.VMEM((K,), jnp.int32),
        pltpu.VMEM((K,), jnp.int32),
        pltpu.VMEM((K,), jnp.int32),
        pltpu.VMEM((K, D), jnp.float32),
        pltpu.VMEM((K, D), jnp.float32),
        pltpu.VMEM((K, D), jnp.float32),
        pltpu.VMEM((K, D), jnp.float32),
        pltpu.SemaphoreType.DMA,
        pltpu.SemaphoreType.DMA,
        pltpu.SemaphoreType.DMA,
        pltpu.SemaphoreType.DMA,
        pltpu.VMEM_SHARED((NP, D), jnp.float32),
    ])

_deg_sc = pl.kernel(
    _deg_body, out_type=_OUT_P, mesh=_SC_MESH,
    scratch_types=[
        pltpu.VMEM((GP * K,), jnp.int32),
        pltpu.VMEM((K,), jnp.int32),
        pltpu.VMEM((K, D), jnp.float32),
        pltpu.VMEM_SHARED((NP, D), jnp.float32),
    ])


def _mm(a, b):
    return lax.dot_general(a, b, (((1,), (0,)), ((), ())),
                           precision=lax.Precision.HIGHEST,
                           preferred_element_type=jnp.float32)


def _acc_stats(i, y, stats_ref):
    @pl.when(i == 0)
    def _():
        stats_ref[...] = jnp.zeros((8, D), jnp.float32)
    stats_ref[0:1, :] += jnp.sum(y, axis=0, keepdims=True)
    stats_ref[1:2, :] += jnp.sum(y * y, axis=0, keepdims=True)


def _sage_a_body(p_ref, pdeg_ref, h_ref, wl_ref, bl_ref, wr_ref,
                 y_ref, stats_ref):
    i = pl.program_id(0)
    s = p_ref[0] + p_ref[1]
    deg = pdeg_ref[0, :, 0:1] + pdeg_ref[1, :, 0:1]
    mean = s / jnp.maximum(deg, 1.0)
    y = _mm(mean, wl_ref[...]) + bl_ref[...] + _mm(h_ref[...], wr_ref[...])
    y_ref[...] = y
    _acc_stats(i, y, stats_ref)


def _gin_a_body(p_ref, h_ref, w1_ref, b1_ref, w2_ref, b2_ref,
                y_ref, stats_ref):
    i = pl.program_id(0)
    hh = h_ref[...] + p_ref[0] + p_ref[1]
    a = jnp.maximum(_mm(hh, w1_ref[...]) + b1_ref[...], 0.0)
    y = _mm(a, w2_ref[...]) + b2_ref[...]
    y_ref[...] = y
    _acc_stats(i, y, stats_ref)


def _bn_b_body(y_ref, stats_ref, g_ref, b_ref, *rest):
    if len(rest) == 2:
        res_ref, o_ref = rest
    else:
        res_ref, (o_ref,) = None, rest
    m = stats_ref[0:1, :] / N
    v = stats_ref[1:2, :] / N - m * m
    z = (y_ref[...] - m) * lax.rsqrt(v + 1e-5) * g_ref[...] + b_ref[...]
    z = jnp.maximum(z, 0.0)
    if res_ref is not None:
        z = z + res_ref[...]
    o_ref[...] = z


def _final_body(p_ref, pdeg_ref, h_ref, wl_ref, bl_ref, wr_ref, o_ref):
    s = p_ref[0] + p_ref[1]
    deg = pdeg_ref[0, :, 0:1] + pdeg_ref[1, :, 0:1]
    mean = s / jnp.maximum(deg, 1.0)
    y = _mm(mean, wl_ref[...]) + bl_ref[...] + _mm(h_ref[...], wr_ref[...])
    mx = jnp.max(y, axis=1, keepdims=True)
    lse = jnp.log(jnp.sum(jnp.exp(y - mx), axis=1, keepdims=True)) + mx
    o_ref[...] = y - lse


_GRID = (N // MB,)
_BLK_P = pl.BlockSpec((NC, MB, D), lambda i: (0, i, 0))
_BLK_H = pl.BlockSpec((MB, D), lambda i: (i, 0))
_BLK_W = pl.BlockSpec((D, D), lambda i: (0, 0))
_BLK_B = pl.BlockSpec((1, D), lambda i: (0, 0))
_BLK_S = pl.BlockSpec((8, D), lambda i: (0, 0))
_OUT_Y = jax.ShapeDtypeStruct((N, D), jnp.float32)
_OUT_S = jax.ShapeDtypeStruct((8, D), jnp.float32)


def _sage_a(p, pdeg, h, wl, bl, wr):
    return pl.pallas_call(
        _sage_a_body, grid=_GRID,
        in_specs=[_BLK_P, _BLK_P, _BLK_H, _BLK_W, _BLK_B, _BLK_W],
        out_specs=[_BLK_H, _BLK_S],
        out_shape=[_OUT_Y, _OUT_S],
    )(p, pdeg, h, wl, bl, wr)


def _gin_a(p, h, w1, b1, w2, b2):
    return pl.pallas_call(
        _gin_a_body, grid=_GRID,
        in_specs=[_BLK_P, _BLK_H, _BLK_W, _BLK_B, _BLK_W, _BLK_B],
        out_specs=[_BLK_H, _BLK_S],
        out_shape=[_OUT_Y, _OUT_S],
    )(p, h, w1, b1, w2, b2)


def _bn_b(y, stats, g, b, res=None):
    in_specs = [_BLK_H, _BLK_S, _BLK_B, _BLK_B]
    args = [y, stats, g, b]
    if res is not None:
        in_specs.append(_BLK_H)
        args.append(res)
    return pl.pallas_call(
        _bn_b_body, grid=_GRID, in_specs=in_specs,
        out_specs=_BLK_H, out_shape=_OUT_Y,
    )(*args)


def _final_tc(p, pdeg, h, wl, bl, wr):
    return pl.pallas_call(
        _final_body, grid=_GRID,
        in_specs=[_BLK_P, _BLK_P, _BLK_H, _BLK_W, _BLK_B, _BLK_W],
        out_specs=_BLK_H, out_shape=_OUT_Y,
    )(p, pdeg, h, wl, bl, wr)


def kernel(x, edge_index, sage0_Wl, sage0_bl, sage0_Wr, gin1_W1, gin1_b1,
           gin1_W2, gin1_b2, sage2_Wl, sage2_bl, sage2_Wr, bn0_g, bn0_b,
           bn1_g, bn1_b, bn2_g, bn2_b, fin_Wl, fin_bl, fin_Wr):
    r = lambda v: v.reshape(1, D)
    zrow = jnp.zeros((RPW, D), jnp.float32)
    ones = jnp.ones((K, D), jnp.float32)
    src = edge_index[0]
    dst = edge_index[1]

    (pdeg,) = _deg_sc(dst, ones, zrow)
    (p,) = _segsum(x, src, dst, zrow)
    y, st = _sage_a(p, pdeg, x, sage0_Wl, r(sage0_bl), sage0_Wr)
    h1 = _bn_b(y, st, r(bn0_g), r(bn0_b))
    (p,) = _segsum(h1, src, dst, zrow)
    y, st = _gin_a(p, h1, gin1_W1, r(gin1_b1), gin1_W2, r(gin1_b2))
    h2 = _bn_b(y, st, r(bn1_g), r(bn1_b), res=h1)
    (p,) = _segsum(h2, src, dst, zrow)
    y, st = _sage_a(p, pdeg, h2, sage2_Wl, r(sage2_bl), sage2_Wr)
    h3 = _bn_b(y, st, r(bn2_g), r(bn2_b), res=h2)
    (p,) = _segsum(h3, src, dst, zrow)
    return _final_tc(p, pdeg, h3, fin_Wl, r(fin_bl), fin_Wr)


# speedup vs baseline: 3.3082x; 1.0092x over previous
import functools

import jax
import jax.numpy as jnp
from jax import lax
from jax.experimental import pallas as pl
from jax.experimental.pallas import tpu as pltpu
from jax.experimental.pallas import tpu_sc as plsc

N = 10000
E = 320000
D = 128
NP = 10240
NC = 2
NS = 16
NW = NC * NS
EPW = E // NW
K = 80
G = EPW // K
PH = 5
GP = G // PH
RPW = NP // NS
DW = 16
DC = 64
MB = 2000


def _segsum_body(h_hbm, src_hbm, dst_hbm, zrow_hbm, out_hbm,
                 idx_s, idx_d, i1sa, i1da, i1sb, i1db, i1sc, i1dc,
                 i1sd, i1dd, rows_a, rows_b, rows_c, rows_d,
                 sem_a, sem_b, sem_c, sem_d, acc):
    cid = lax.axis_index("c")
    sid = lax.axis_index("s")
    wid = cid * NS + sid
    r0 = sid * RPW

    pltpu.sync_copy(zrow_hbm, acc.at[pl.ds(r0, RPW)])
    plsc.subcore_barrier()

    def copyidx(g, i1s, i1d):
        for c in range(K // 16):
            i1s[pl.ds(16 * c, 16)] = idx_s[pl.ds(g * K + 16 * c, 16)]
            i1d[pl.ds(16 * c, 16)] = idx_d[pl.ds(g * K + 16 * c, 16)]

    def startg(i1s, rows, sem):
        pltpu.make_async_copy(h_hbm.at[i1s], rows, sem).start()

    def waitg(i1s, rows, sem):
        pltpu.make_async_copy(h_hbm.at[i1s], rows, sem).wait()

    def scat(i1d, rows):
        pltpu.sync_copy(rows, acc.at[i1d], add=True)

    for ph in range(PH):
        base = wid * EPW + ph * (GP * K)
        pltpu.sync_copy(src_hbm.at[pl.ds(base, GP * K)], idx_s)
        pltpu.sync_copy(dst_hbm.at[pl.ds(base, GP * K)], idx_d)

        bufs = [(i1sa, i1da, rows_a, sem_a),
                (i1sb, i1db, rows_b, sem_b),
                (i1sc, i1dc, rows_c, sem_c),
                (i1sd, i1dd, rows_d, sem_d)]
        for j in range(3):
            copyidx(j, bufs[j][0], bufs[j][1])
            startg(bufs[j][0], bufs[j][2], bufs[j][3])

        @pl.loop(0, GP, step=4)
        def _(g0):
            for j in range(4):
                cs, cd, cr, csem = bufs[j]
                ns, nd, nr, nsem = bufs[(j + 3) % 4]
                g = g0 + j

                @pl.when(g < GP)
                def _():
                    waitg(cs, cr, csem)

                    @pl.when(g + 3 < GP)
                    def _():
                        copyidx(g + 3, ns, nd)
                        startg(ns, nr, nsem)

                    scat(cd, cr)

    plsc.subcore_barrier()
    pltpu.sync_copy(acc.at[pl.ds(r0, RPW)], out_hbm.at[cid, pl.ds(r0, RPW)])


def _deg_body(dst_hbm, ones_hbm, zrow_hbm, out_hbm, idx_d, i1da, i1db,
              ones_v, sem_a, sem_b, acc):
    cid = lax.axis_index("c")
    sid = lax.axis_index("s")
    wid = cid * NS + sid
    r0 = sid * RPW

    pltpu.sync_copy(zrow_hbm, acc.at[pl.ds(r0, RPW)])
    pltpu.sync_copy(ones_hbm, ones_v)
    plsc.subcore_barrier()

    def copyidx(g, i1d):
        for c in range(K // 16):
            i1d[pl.ds(16 * c, 16)] = idx_d[pl.ds(g * K + 16 * c, 16)]

    def starts(i1d, sem):
        pltpu.make_async_copy(ones_v, acc.at[i1d], sem).start()

    def waits(i1d, sem):
        pltpu.make_async_copy(ones_v, acc.at[i1d], sem).wait()

    for ph in range(PH):
        base = wid * EPW + ph * (GP * K)
        pltpu.sync_copy(dst_hbm.at[pl.ds(base, GP * K)], idx_d)

        @pl.loop(0, GP, step=2)
        def _(g):
            @pl.when(g >= 2)
            def _():
                waits(i1da, sem_a)

            copyidx(g, i1da)
            starts(i1da, sem_a)

            @pl.when(g + 1 < GP)
            def _():
                @pl.when(g >= 2)
                def _():
                    waits(i1db, sem_b)

                copyidx(g + 1, i1db)
                starts(i1db, sem_b)

        waits(i1da, sem_a)
        waits(i1db, sem_b)

    plsc.subcore_barrier()
    pltpu.sync_copy(acc.at[pl.ds(r0, RPW)], out_hbm.at[cid, pl.ds(r0, RPW)])


_SC_MESH = plsc.VectorSubcoreMesh(core_axis_name="c", subcore_axis_name="s")
_OUT_P = [jax.ShapeDtypeStruct((NC, NP, D), jnp.float32)]

_segsum = pl.kernel(
    _segsum_body, out_type=_OUT_P, mesh=_SC_MESH,
    scratch_types=[
        pltpu.VMEM((GP * K,), jnp.int32),
        pltpu.VMEM((GP * K,), jnp.int32),
        pltpu.VMEM((K,), jnp.int32),
        pltpu.VMEM((K,), jnp.int32),
        pltpu.VMEM((K,), jnp.int32),
        pltpu.VMEM((K,), jnp.int32),
        pltpu.VMEM((K,), jnp.int32),
        pltpu.VMEM((K,), jnp.int32),
        pltpu.VMEM((K,), jnp.int32),
        pltpu.VMEM((K,), jnp.int32),
        pltpu.VMEM((K, D), jnp.float32),
        pltpu.VMEM((K, D), jnp.float32),
        pltpu.VMEM((K, D), jnp.float32),
        pltpu.VMEM((K, D), jnp.float32),
        pltpu.SemaphoreType.DMA,
        pltpu.SemaphoreType.DMA,
        pltpu.SemaphoreType.DMA,
        pltpu.SemaphoreType.DMA,
        pltpu.VMEM_SHARED((NP, D), jnp.float32),
    ])

_deg_sc = pl.kernel(
    _deg_body, out_type=_OUT_P, mesh=_SC_MESH,
    scratch_types=[
        pltpu.VMEM((GP * K,), jnp.int32),
        pltpu.VMEM((K,), jnp.int32),
        pltpu.VMEM((K,), jnp.int32),
        pltpu.VMEM((K, D), jnp.float32),
        pltpu.SemaphoreType.DMA,
        pltpu.SemaphoreType.DMA,
        pltpu.VMEM_SHARED((NP, D), jnp.float32),
    ])


def _mm(a, b):
    return lax.dot_general(a, b, (((1,), (0,)), ((), ())),
                           precision=lax.Precision.HIGHEST,
                           preferred_element_type=jnp.float32)


def _acc_stats(i, y, stats_ref):
    @pl.when(i == 0)
    def _():
        stats_ref[...] = jnp.zeros((8, D), jnp.float32)
    stats_ref[0:1, :] += jnp.sum(y, axis=0, keepdims=True)
    stats_ref[1:2, :] += jnp.sum(y * y, axis=0, keepdims=True)


def _sage_a_body(p_ref, pdeg_ref, h_ref, wl_ref, bl_ref, wr_ref,
                 y_ref, stats_ref):
    i = pl.program_id(0)
    s = p_ref[0] + p_ref[1]
    deg = pdeg_ref[0, :, 0:1] + pdeg_ref[1, :, 0:1]
    mean = s / jnp.maximum(deg, 1.0)
    y = _mm(mean, wl_ref[...]) + bl_ref[...] + _mm(h_ref[...], wr_ref[...])
    y_ref[...] = y
    _acc_stats(i, y, stats_ref)


def _gin_a_body(p_ref, h_ref, w1_ref, b1_ref, w2_ref, b2_ref,
                y_ref, stats_ref):
    i = pl.program_id(0)
    hh = h_ref[...] + p_ref[0] + p_ref[1]
    a = jnp.maximum(_mm(hh, w1_ref[...]) + b1_ref[...], 0.0)
    y = _mm(a, w2_ref[...]) + b2_ref[...]
    y_ref[...] = y
    _acc_stats(i, y, stats_ref)


def _bn_b_body(y_ref, stats_ref, g_ref, b_ref, *rest):
    if len(rest) == 2:
        res_ref, o_ref = rest
    else:
        res_ref, (o_ref,) = None, rest
    m = stats_ref[0:1, :] / N
    v = stats_ref[1:2, :] / N - m * m
    z = (y_ref[...] - m) * lax.rsqrt(v + 1e-5) * g_ref[...] + b_ref[...]
    z = jnp.maximum(z, 0.0)
    if res_ref is not None:
        z = z + res_ref[...]
    o_ref[...] = z


def _final_body(p_ref, pdeg_ref, h_ref, wl_ref, bl_ref, wr_ref, o_ref):
    s = p_ref[0] + p_ref[1]
    deg = pdeg_ref[0, :, 0:1] + pdeg_ref[1, :, 0:1]
    mean = s / jnp.maximum(deg, 1.0)
    y = _mm(mean, wl_ref[...]) + bl_ref[...] + _mm(h_ref[...], wr_ref[...])
    mx = jnp.max(y, axis=1, keepdims=True)
    lse = jnp.log(jnp.sum(jnp.exp(y - mx), axis=1, keepdims=True)) + mx
    o_ref[...] = y - lse


_GRID = (N // MB,)
_BLK_P = pl.BlockSpec((NC, MB, D), lambda i: (0, i, 0))
_BLK_H = pl.BlockSpec((MB, D), lambda i: (i, 0))
_BLK_W = pl.BlockSpec((D, D), lambda i: (0, 0))
_BLK_B = pl.BlockSpec((1, D), lambda i: (0, 0))
_BLK_S = pl.BlockSpec((8, D), lambda i: (0, 0))
_OUT_Y = jax.ShapeDtypeStruct((N, D), jnp.float32)
_OUT_S = jax.ShapeDtypeStruct((8, D), jnp.float32)


def _sage_a(p, pdeg, h, wl, bl, wr):
    return pl.pallas_call(
        _sage_a_body, grid=_GRID,
        in_specs=[_BLK_P, _BLK_P, _BLK_H, _BLK_W, _BLK_B, _BLK_W],
        out_specs=[_BLK_H, _BLK_S],
        out_shape=[_OUT_Y, _OUT_S],
    )(p, pdeg, h, wl, bl, wr)


def _gin_a(p, h, w1, b1, w2, b2):
    return pl.pallas_call(
        _gin_a_body, grid=_GRID,
        in_specs=[_BLK_P, _BLK_H, _BLK_W, _BLK_B, _BLK_W, _BLK_B],
        out_specs=[_BLK_H, _BLK_S],
        out_shape=[_OUT_Y, _OUT_S],
    )(p, h, w1, b1, w2, b2)


def _bn_b(y, stats, g, b, res=None):
    in_specs = [_BLK_H, _BLK_S, _BLK_B, _BLK_B]
    args = [y, stats, g, b]
    if res is not None:
        in_specs.append(_BLK_H)
        args.append(res)
    return pl.pallas_call(
        _bn_b_body, grid=_GRID, in_specs=in_specs,
        out_specs=_BLK_H, out_shape=_OUT_Y,
    )(*args)


def _final_tc(p, pdeg, h, wl, bl, wr):
    return pl.pallas_call(
        _final_body, grid=_GRID,
        in_specs=[_BLK_P, _BLK_P, _BLK_H, _BLK_W, _BLK_B, _BLK_W],
        out_specs=_BLK_H, out_shape=_OUT_Y,
    )(p, pdeg, h, wl, bl, wr)


def kernel(x, edge_index, sage0_Wl, sage0_bl, sage0_Wr, gin1_W1, gin1_b1,
           gin1_W2, gin1_b2, sage2_Wl, sage2_bl, sage2_Wr, bn0_g, bn0_b,
           bn1_g, bn1_b, bn2_g, bn2_b, fin_Wl, fin_bl, fin_Wr):
    r = lambda v: v.reshape(1, D)
    zrow = jnp.zeros((RPW, D), jnp.float32)
    ones = jnp.ones((K, D), jnp.float32)
    src = edge_index[0]
    dst = edge_index[1]

    (pdeg,) = _deg_sc(dst, ones, zrow)
    (p,) = _segsum(x, src, dst, zrow)
    y, st = _sage_a(p, pdeg, x, sage0_Wl, r(sage0_bl), sage0_Wr)
    h1 = _bn_b(y, st, r(bn0_g), r(bn0_b))
    (p,) = _segsum(h1, src, dst, zrow)
    y, st = _gin_a(p, h1, gin1_W1, r(gin1_b1), gin1_W2, r(gin1_b2))
    h2 = _bn_b(y, st, r(bn1_g), r(bn1_b), res=h1)
    (p,) = _segsum(h2, src, dst, zrow)
    y, st = _sage_a(p, pdeg, h2, sage2_Wl, r(sage2_bl), sage2_Wr)
    h3 = _bn_b(y, st, r(bn2_g), r(bn2_b), res=h2)
    (p,) = _segsum(h3, src, dst, zrow)
    return _final_tc(p, pdeg, h3, fin_Wl, r(fin_bl), fin_Wr)
